# baseline (speedup 1.0000x reference)
import jax
import jax.numpy as jnp
from jax.experimental import pallas as pl


_BLOCK_S = 512


def _copy_kernel(table_ref, out_ref):
    out_ref[...] = jnp.broadcast_to(table_ref[...][None], out_ref.shape)


def kernel(x, table):
    B, S, D = x.shape
    grid = (S // _BLOCK_S,)
    return pl.pallas_call(
        _copy_kernel,
        grid=grid,
        in_specs=[
            pl.BlockSpec((_BLOCK_S, D), lambda s: (s, 0)),
        ],
        out_specs=pl.BlockSpec((B, _BLOCK_S, D), lambda s: (0, s, 0)),
        out_shape=jax.ShapeDtypeStruct((B, S, D), table.dtype),
    )(table[:S])
